# Initial kernel scaffold; baseline (speedup 1.0000x reference)
import jax
import jax.numpy as jnp
from jax.experimental import pallas as pl


def kernel(x, edge_index, W1, b1, W2, b2, bn_gamma, bn_beta, bn_mean, bn_var):
    raise NotImplementedError("write your pallas kernel here")


# speedup vs baseline: 5.6975x; 5.6975x over previous
import functools

import jax
import jax.numpy as jnp
from jax import lax
from jax.experimental import pallas as pl
from jax.experimental.pallas import tpu as pltpu
from jax.experimental.pallas import tpu_sc as plsc

N = 10000
E = 320000
NC = 2
NS = 16
CHUNK = 128
CH = -(-E // (NS * CHUNK))
EPT = CH * CHUNK
PAD = NS * EPT - E
ACC_ROWS = 10112
ZCH = 128
ZPT = ACC_ROWS // NS
WPT = 624
WTAIL = N - NS * WPT


def _make_agg(F):
    mesh = plsc.VectorSubcoreMesh(core_axis_name="c", subcore_axis_name="s")

    @functools.partial(
        pl.kernel,
        out_type=jax.ShapeDtypeStruct((NC, N, F), jnp.float32),
        mesh=mesh,
        compiler_params=pltpu.CompilerParams(use_tc_tiling_on_sc=False),
        scratch_types=[
            pltpu.VMEM((CH, CHUNK), jnp.int32),
            pltpu.VMEM((CH, CHUNK), jnp.int32),
            pltpu.VMEM((CHUNK, F), jnp.float32),
            pltpu.VMEM((ZCH, F), jnp.float32),
            pltpu.VMEM_SHARED((ACC_ROWS, F), jnp.float32),
        ],
    )
    def agg(table_hbm, srcs_hbm, dsts_hbm, zeros_hbm, out_hbm, si, di, rows, zb, acc):
        c = lax.axis_index("c")
        s = lax.axis_index("s")

        pltpu.sync_copy(zeros_hbm, zb)
        zbase = s * ZPT
        for k in range(ZPT // ZCH):
            pltpu.sync_copy(zb, acc.at[pl.ds(zbase + k * ZCH, ZCH)])
        rem = ZPT % ZCH
        if rem:
            pltpu.sync_copy(zb.at[pl.ds(0, rem)],
                            acc.at[pl.ds(zbase + (ZPT // ZCH) * ZCH, rem)])
        plsc.subcore_barrier()

        pltpu.sync_copy(srcs_hbm.at[c * NS + s], si)
        pltpu.sync_copy(dsts_hbm.at[s], di)

        @pl.loop(0, CH)
        def _(j):
            pltpu.sync_copy(table_hbm.at[si.at[j]], rows)
            pltpu.sync_copy(rows, acc.at[di.at[j]], add=True)

        plsc.subcore_barrier()

        wbase = s * WPT
        pltpu.sync_copy(acc.at[pl.ds(wbase, WPT)],
                        out_hbm.at[c].at[pl.ds(wbase, WPT)])

        @pl.when(s == NS - 1)
        def _():
            pltpu.sync_copy(acc.at[pl.ds(NS * WPT, WTAIL)],
                            out_hbm.at[c].at[pl.ds(NS * WPT, WTAIL)])

    return agg


_agg64 = _make_agg(64)
_agg32 = _make_agg(32)

_BR = 1000


def _fused_body(p_ref, w1_ref, c1_ref, w2_ref, o_ref):
    h = jnp.concatenate([p_ref[0], p_ref[1]], axis=1)
    h = jnp.dot(h, w1_ref[...], preferred_element_type=jnp.float32)
    h = jnp.maximum(h + c1_ref[...], 0.0)
    y = jnp.dot(h, w2_ref[...], preferred_element_type=jnp.float32)
    o_ref[0] = y[:, :32]
    o_ref[1] = y[:, 32:]


_fused = pl.pallas_call(
    _fused_body,
    grid=(N // _BR,),
    in_specs=[
        pl.BlockSpec((NC, _BR, 64), lambda i: (0, i, 0)),
        pl.BlockSpec((128, 128), lambda i: (0, 0)),
        pl.BlockSpec((1, 128), lambda i: (0, 0)),
        pl.BlockSpec((128, 64), lambda i: (0, 0)),
    ],
    out_specs=pl.BlockSpec((NC, _BR, 32), lambda i: (0, i, 0)),
    out_shape=jax.ShapeDtypeStruct((NC, N, 32), jnp.float32),
)


def _final_body(q_ref, b2_ref, o_ref):
    v = jnp.concatenate([q_ref[0], q_ref[1]], axis=1) + b2_ref[...]
    nrm = jnp.sqrt(jnp.sum(v * v, axis=1, keepdims=True))
    o_ref[...] = v / jnp.maximum(nrm, 1e-12)


_final = pl.pallas_call(
    _final_body,
    grid=(N // _BR,),
    in_specs=[
        pl.BlockSpec((NC, _BR, 32), lambda i: (0, i, 0)),
        pl.BlockSpec((1, 64), lambda i: (0, 0)),
    ],
    out_specs=pl.BlockSpec((_BR, 64), lambda i: (i, 0)),
    out_shape=jax.ShapeDtypeStruct((N, 64), jnp.float32),
)


def kernel(x, edge_index, W1, b1, W2, b2, bn_gamma, bn_beta, bn_mean, bn_var):
    scale = bn_gamma * lax.rsqrt(bn_var + 1e-5)
    W1e = W1 * scale[None, :]
    c1 = ((b1 - bn_mean) * scale + bn_beta)[None, :]

    src = jnp.concatenate([edge_index[0], jnp.zeros((PAD,), jnp.int32)])
    dst = jnp.concatenate([edge_index[1], jnp.full((PAD,), N, jnp.int32)])
    srcs1 = src.reshape(1, NS, CH, CHUNK)
    srcs = jnp.concatenate([srcs1, srcs1 + N]).reshape(NC * NS, CH, CHUNK)
    dsts = dst.reshape(NS, CH, CHUNK)

    z64 = jnp.zeros((ZCH, 64), jnp.float32)
    z32 = jnp.zeros((ZCH, 32), jnp.float32)

    x2 = jnp.moveaxis(x.reshape(N, 2, 64), 1, 0).reshape(2 * N, 64)

    p = _agg64(x2, srcs, dsts, z64)
    h2 = _fused(p, W1e, c1, W2)
    q = _agg32(h2.reshape(2 * N, 32), srcs, dsts, z32)
    return _final(q, b2[None, :])
